# baseline (speedup 1.0000x reference)
import functools

import jax
import jax.numpy as jnp
from jax import lax
from jax.experimental import pallas as pl
from jax.experimental.pallas import tpu as pltpu
from jax.experimental.pallas import tpu_sc as plsc

B, L, C = 4096, 200, 128
VOCAB = 262
NUM_CORES = 2
NUM_SUBCORES = 16
NW = NUM_CORES * NUM_SUBCORES
RPW = B // NW
NBUF = 2
NGX = RPW // NBUF
NGP = L // NBUF
G0, G1 = 128, L - 128
PCH = 48
PCHUNKS = ((0, 48), (48, 48), (96, 48), (144, 48), (192, 48), (240, 22))


def _sc_body(idx_hbm, table_hbm, tab_ev_hbm, tab_od_hbm, x_hbm, pt_hbm,
             pooled_hbm, pa_v, pb_v, pc_v, idxblk_v, xrows_v,
             icol_v, gb0_v, gb1_v, tb0_v, tb1_v, sem_g, sem_o):
    c = lax.axis_index("c")
    s = lax.axis_index("s")
    wid = s * NUM_CORES + c

    @pl.when(s == 0)
    def _():
        for off, nrows in PCHUNKS:
            rows = pl.ds(off, nrows)
            pltpu.sync_copy(tab_ev_hbm.at[rows], pa_v.at[pl.ds(0, nrows)])
            pltpu.sync_copy(tab_od_hbm.at[rows], pb_v.at[pl.ds(0, nrows)])

            def row_body(r, carry):
                for j in range(4):
                    sl = pl.ds(j * 16, 16)
                    pc_v[r, sl] = jnp.maximum(pa_v[r, sl], pb_v[r, sl])
                return carry

            lax.fori_loop(0, nrows, row_body, 0)
            pltpu.sync_copy(pc_v.at[pl.ds(0, nrows)], pooled_hbm.at[rows])

    plsc.subcore_barrier()

    base_w = wid * RPW
    pltpu.sync_copy(idx_hbm.at[pl.ds(base_w, RPW)], idxblk_v)

    def start_all(copies):
        for cp in copies:
            cp.start()

    def wait_all(copies):
        for cp in copies:
            cp.wait()

    def g_copies(i, k):
        return (
            pltpu.make_async_copy(
                table_hbm.at[idxblk_v.at[i, pl.ds(0, G0)]],
                xrows_v.at[k, pl.ds(0, G0)], sem_g.at[k]),
            pltpu.make_async_copy(
                table_hbm.at[idxblk_v.at[i, pl.ds(G0, G1)]],
                xrows_v.at[k, pl.ds(G0, G1)], sem_g.at[k]),
        )

    def o_copy(i, k):
        return pltpu.make_async_copy(
            xrows_v.at[k], x_hbm.at[base_w + i], sem_o.at[k])

    for k in range(NBUF):
        start_all(g_copies(k, k))

    def ring_body(g, carry):
        for k in range(NBUF):
            i = g * NBUF + k
            wait_all(g_copies(i, k))
            o_copy(i, k).start()

            @pl.when(g < NGX - 1)
            def _():
                o_copy(i, k).wait()
                start_all(g_copies(i + NBUF, k))
        return carry

    lax.fori_loop(0, NGX, ring_body, 0)
    for k in range(NBUF):
        o_copy((NGX - 1) * NBUF + k, k).wait()

    gbufs = (gb0_v, gb1_v)
    tbufs = (tb0_v, tb1_v)
    row_ids = [lax.iota(jnp.int32, 16) + 16 * t for t in range(8)]

    def fill_icol(i, k):
        col = jnp.zeros((16,), jnp.int32) + i
        for t in range(8):
            v = plsc.load_gather(idxblk_v, [row_ids[t], col])
            icol_v[k, pl.ds(16 * t, 16)] = v

    def pg_copy(k):
        return pltpu.make_async_copy(
            pooled_hbm.at[icol_v.at[k]], gbufs[k], sem_g.at[k])

    def po_copy(i, k):
        return pltpu.make_async_copy(
            tbufs[k], pt_hbm.at[i, :, wid], sem_o.at[k])

    def transpose_block(k):
        gb, tb = gbufs[k], tbufs[k]

        def cb_body(cb, carry):
            for ci in range(8):
                col = jnp.zeros((16,), jnp.int32) + (cb * 8 + ci)
                for t in range(8):
                    v = plsc.load_gather(gb, [row_ids[t], col])
                    tb[cb, ci, pl.ds(16 * t, 16)] = v
            return carry

        lax.fori_loop(0, 8, cb_body, 0)

    for k in range(NBUF):
        fill_icol(k, k)
        pg_copy(k).start()

    def pring_body(g, carry):
        for k in range(NBUF):
            i = g * NBUF + k
            pg_copy(k).wait()

            @pl.when(g > 0)
            def _():
                po_copy(i - NBUF, k).wait()
            transpose_block(k)
            po_copy(i, k).start()

            @pl.when(g < NGP - 1)
            def _():
                fill_icol(i + NBUF, k)
                pg_copy(k).start()
        return carry

    lax.fori_loop(0, NGP, pring_body, 0)
    for k in range(NBUF):
        po_copy((NGP - 1) * NBUF + k, k).wait()


@jax.jit
def kernel(inputs, table):
    idx = inputs.astype(jnp.int32)
    table = table.astype(jnp.float32)
    tab_ev = table[:, 0::2]
    tab_od = table[:, 1::2]
    mesh = plsc.VectorSubcoreMesh(core_axis_name="c", subcore_axis_name="s")
    call = pl.kernel(
        _sc_body,
        mesh=mesh,
        compiler_params=pltpu.CompilerParams(
            use_tc_tiling_on_sc=False, needs_layout_passes=False),
        out_type=[
            jax.ShapeDtypeStruct((B, L, C), jnp.float32),
            jax.ShapeDtypeStruct((L, 8, B // 128, 8, 128), jnp.float32),
            jax.ShapeDtypeStruct((VOCAB, C // 2), jnp.float32),
        ],
        scratch_types=[
            pltpu.VMEM((PCH, C // 2), jnp.float32),
            pltpu.VMEM((PCH, C // 2), jnp.float32),
            pltpu.VMEM((PCH, C // 2), jnp.float32),
            pltpu.VMEM((RPW, L), jnp.int32),
            pltpu.VMEM((NBUF, L, C), jnp.float32),
            pltpu.VMEM((NBUF, RPW), jnp.int32),
            pltpu.VMEM((RPW, C // 2), jnp.float32),
            pltpu.VMEM((RPW, C // 2), jnp.float32),
            pltpu.VMEM((8, 8, RPW), jnp.float32),
            pltpu.VMEM((8, 8, RPW), jnp.float32),
            pltpu.SemaphoreType.DMA((NBUF,)),
            pltpu.SemaphoreType.DMA((NBUF,)),
        ],
    )
    x, pt, _pooled = call(idx, table, tab_ev, tab_od)
    p = jnp.transpose(pt, (2, 4, 0, 1, 3)).reshape(B, L, C // 2)
    return (x, p)


# speedup vs baseline: 1.6010x; 1.6010x over previous
import functools

import jax
import jax.numpy as jnp
from jax import lax
from jax.experimental import pallas as pl
from jax.experimental.pallas import tpu as pltpu
from jax.experimental.pallas import tpu_sc as plsc

B, L, C = 4096, 200, 128
VOCAB = 262
NUM_CORES = 2
NUM_SUBCORES = 16
NW = NUM_CORES * NUM_SUBCORES
RPW = B // NW
NBUF = 2
NGX = RPW // NBUF
NGP = L // NBUF
G0, G1 = 128, L - 128
PCH = 48
PCHUNKS = ((0, 48), (48, 48), (96, 48), (144, 48), (192, 48), (240, 22))


def _sc_body(idx_hbm, table_hbm, tab_ev_hbm, tab_od_hbm, x_hbm, pt_hbm,
             pooled_hbm, pa_v, pb_v, pc_v, idxblk_v, xrows_v,
             icol_v, gb0_v, gb1_v, tb0_v, tb1_v, sem_g, sem_o):
    c = lax.axis_index("c")
    s = lax.axis_index("s")
    wid = s * NUM_CORES + c

    @pl.when(s == 0)
    def _():
        for off, nrows in PCHUNKS:
            rows = pl.ds(off, nrows)
            pltpu.sync_copy(tab_ev_hbm.at[rows], pa_v.at[pl.ds(0, nrows)])
            pltpu.sync_copy(tab_od_hbm.at[rows], pb_v.at[pl.ds(0, nrows)])

            def row_body(r, carry):
                for j in range(4):
                    sl = pl.ds(j * 16, 16)
                    pc_v[r, sl] = jnp.maximum(pa_v[r, sl], pb_v[r, sl])
                return carry

            lax.fori_loop(0, nrows, row_body, 0)
            pltpu.sync_copy(pc_v.at[pl.ds(0, nrows)], pooled_hbm.at[rows])

    plsc.subcore_barrier()

    base_w = wid * RPW
    pltpu.sync_copy(idx_hbm.at[pl.ds(base_w, RPW)], idxblk_v)

    def start_all(copies):
        for cp in copies:
            cp.start()

    def wait_all(copies):
        for cp in copies:
            cp.wait()

    def g_copies(i, k):
        return (
            pltpu.make_async_copy(
                table_hbm.at[idxblk_v.at[i, pl.ds(0, G0)]],
                xrows_v.at[k, pl.ds(0, G0)], sem_g.at[k]),
            pltpu.make_async_copy(
                table_hbm.at[idxblk_v.at[i, pl.ds(G0, G1)]],
                xrows_v.at[k, pl.ds(G0, G1)], sem_g.at[k]),
        )

    def o_copy(i, k):
        return pltpu.make_async_copy(
            xrows_v.at[k], x_hbm.at[base_w + i], sem_o.at[k])

    for k in range(NBUF):
        start_all(g_copies(k, k))

    def ring_body(g, carry):
        for k in range(NBUF):
            i = g * NBUF + k
            wait_all(g_copies(i, k))
            o_copy(i, k).start()

            @pl.when(g < NGX - 1)
            def _():
                o_copy(i, k).wait()
                start_all(g_copies(i + NBUF, k))
        return carry

    lax.fori_loop(0, NGX, ring_body, 0)
    for k in range(NBUF):
        o_copy((NGX - 1) * NBUF + k, k).wait()

    gbufs = (gb0_v, gb1_v)
    tbufs = (tb0_v, tb1_v)
    row_ids = [lax.iota(jnp.int32, 16) + 16 * t for t in range(8)]

    def fill_icol(i, k):
        col = jnp.zeros((16,), jnp.int32) + i
        for t in range(8):
            v = plsc.load_gather(idxblk_v, [row_ids[t], col])
            icol_v[k, pl.ds(16 * t, 16)] = v

    def pg_copy(k):
        return pltpu.make_async_copy(
            pooled_hbm.at[icol_v.at[k]], gbufs[k], sem_g.at[k])

    def po_copy(i, k):
        return pltpu.make_async_copy(
            tbufs[k].at[:, :, pl.ds(0, RPW)], pt_hbm.at[i, :, wid],
            sem_o.at[k])

    iota16 = lax.iota(jnp.int32, 16)
    cb_idx = [(16 * q + iota16) // 8 for q in range(4)]
    ci_idx = [(16 * q + iota16) % 8 for q in range(4)]

    def transpose_block(k):
        gb, tb = gbufs[k], tbufs[k]

        def r_body(r, carry):
            bvec = jnp.zeros((16,), jnp.int32) + r
            for q in range(4):
                v = gb[r, pl.ds(16 * q, 16)]
                plsc.store_scatter(tb, [cb_idx[q], ci_idx[q], bvec], v)
            return carry

        lax.fori_loop(0, RPW, r_body, 0)

    for k in range(NBUF):
        fill_icol(k, k)
        pg_copy(k).start()

    def pring_body(g, carry):
        for k in range(NBUF):
            i = g * NBUF + k
            pg_copy(k).wait()

            @pl.when(g > 0)
            def _():
                po_copy(i - NBUF, k).wait()
            transpose_block(k)
            po_copy(i, k).start()

            @pl.when(g < NGP - 1)
            def _():
                fill_icol(i + NBUF, k)
                pg_copy(k).start()
        return carry

    lax.fori_loop(0, NGP, pring_body, 0)
    for k in range(NBUF):
        po_copy((NGP - 1) * NBUF + k, k).wait()


@jax.jit
def kernel(inputs, table):
    idx = inputs.astype(jnp.int32)
    table = table.astype(jnp.float32)
    tab_ev = table[:, 0::2]
    tab_od = table[:, 1::2]
    mesh = plsc.VectorSubcoreMesh(core_axis_name="c", subcore_axis_name="s")
    call = pl.kernel(
        _sc_body,
        mesh=mesh,
        compiler_params=pltpu.CompilerParams(
            use_tc_tiling_on_sc=False, needs_layout_passes=False),
        out_type=[
            jax.ShapeDtypeStruct((B, L, C), jnp.float32),
            jax.ShapeDtypeStruct((L, 8, B // 128, 8, 128), jnp.float32),
            jax.ShapeDtypeStruct((VOCAB, C // 2), jnp.float32),
        ],
        scratch_types=[
            pltpu.VMEM((PCH, C // 2), jnp.float32),
            pltpu.VMEM((PCH, C // 2), jnp.float32),
            pltpu.VMEM((PCH, C // 2), jnp.float32),
            pltpu.VMEM((RPW, L), jnp.int32),
            pltpu.VMEM((NBUF, L, C), jnp.float32),
            pltpu.VMEM((NBUF, RPW), jnp.int32),
            pltpu.VMEM((RPW, C // 2), jnp.float32),
            pltpu.VMEM((RPW, C // 2), jnp.float32),
            pltpu.VMEM((8, 8, RPW + 1), jnp.float32),
            pltpu.VMEM((8, 8, RPW + 1), jnp.float32),
            pltpu.SemaphoreType.DMA((NBUF,)),
            pltpu.SemaphoreType.DMA((NBUF,)),
        ],
    )
    x, pt, _pooled = call(idx, table, tab_ev, tab_od)
    p = jnp.transpose(pt, (2, 4, 0, 1, 3)).reshape(B, L, C // 2)
    return (x, p)
